# baseline (speedup 1.0000x reference)
import functools

import jax
import jax.numpy as jnp
import numpy as np
from jax import lax
from jax.experimental import pallas as pl
from jax.experimental.pallas import tpu as pltpu
from jax.experimental.pallas import tpu_sc as plsc

B = 16384
D = 64
BN_EPS = 1e-5

NC = 2
NS = 16
NW = NC * NS
BPW = B // NW
CHUNK = 128
NCH = BPW // CHUNK

@functools.lru_cache(maxsize=None)
def _make_sc_gather():
    mesh = plsc.VectorSubcoreMesh(core_axis_name="c", subcore_axis_name="s")

    @functools.partial(
        pl.kernel,
        mesh=mesh,
        compiler_params=pltpu.CompilerParams(use_tc_tiling_on_sc=False),
        out_type=[
            jax.ShapeDtypeStruct((NW, NCH, CHUNK, D), jnp.float32),
            jax.ShapeDtypeStruct((NW, NCH, CHUNK, D), jnp.float32),
        ],
        scratch_types=[
            pltpu.VMEM((NCH, CHUNK), jnp.int32),
            pltpu.VMEM((NCH, CHUNK), jnp.int32),
            pltpu.VMEM((NCH, CHUNK, D), jnp.float32),
            pltpu.VMEM((NCH, CHUNK, D), jnp.float32),
            pltpu.SemaphoreType.DMA,
        ],
    )
    def _sc_gather(users_hbm, movies_hbm, ut_hbm, mt_hbm, ue_hbm, me_hbm,
                   idx_u, idx_m, rows_u, rows_m, sem):
        wid = lax.axis_index("s") * NC + lax.axis_index("c")
        pltpu.sync_copy(users_hbm.at[wid], idx_u)
        pltpu.sync_copy(movies_hbm.at[wid], idx_m)
        copies = []
        for j in range(NCH):
            copies.append(pltpu.async_copy(ut_hbm.at[idx_u.at[j]], rows_u.at[j], sem))
            copies.append(pltpu.async_copy(mt_hbm.at[idx_m.at[j]], rows_m.at[j], sem))
        for c in copies:
            c.wait()
        pltpu.sync_copy(rows_u, ue_hbm.at[wid])
        pltpu.sync_copy(rows_m, me_hbm.at[wid])

    return _sc_gather


def _mlp_body(ue_ref, me_ref, w1_ref, c1_ref, w2_ref, c2_ref, w3_ref, c3_ref,
              out_ref):
    w1 = w1_ref[...]
    nt = (((1,), (1,)), ((), ()))
    h = lax.dot_general(ue_ref[...], w1[:, :D], nt,
                        preferred_element_type=jnp.float32)
    h += lax.dot_general(me_ref[...], w1[:, D:], nt,
                         preferred_element_type=jnp.float32)
    h = jnp.maximum(h + c1_ref[...], 0.0)
    h = lax.dot_general(h, w2_ref[...], nt, preferred_element_type=jnp.float32)
    h = jnp.maximum(h + c2_ref[...], 0.0)
    o = jnp.sum(h * w3_ref[...], axis=1, keepdims=True) + c3_ref[...]
    out_ref[...] = 1.0 / (1.0 + jnp.exp(-o))


def kernel(users, movies, user_table, movie_table,
           W1, b1, g1, be1, W2, b2, g2, be2, W3, b3):
    u = users.astype(jnp.int32).reshape(NW, NCH, CHUNK)
    m = movies.astype(jnp.int32).reshape(NW, NCH, CHUNK)
    ue4, me4 = _make_sc_gather()(u, m, user_table, movie_table)
    ue = ue4.reshape(B, D)
    me = me4.reshape(B, D)

    s = np.float32(1.0 / np.sqrt(1.0 + BN_EPS))
    w1 = W1 * (g1 * s)[:, None]
    c1 = (b1 * g1 * s + be1).reshape(1, 32)
    w2 = W2 * (g2 * s)[:, None]
    c2 = (b2 * g2 * s + be2).reshape(1, 16)
    w3 = W3.reshape(1, 16)
    c3 = b3.reshape(1, 1)

    R = 2048
    NB = B // R
    out = pl.pallas_call(
        _mlp_body,
        grid=(NB,),
        in_specs=[
            pl.BlockSpec((R, D), lambda i: (i, 0)),
            pl.BlockSpec((R, D), lambda i: (i, 0)),
            pl.BlockSpec((32, 128), lambda i: (0, 0)),
            pl.BlockSpec((1, 32), lambda i: (0, 0)),
            pl.BlockSpec((16, 32), lambda i: (0, 0)),
            pl.BlockSpec((1, 16), lambda i: (0, 0)),
            pl.BlockSpec((1, 16), lambda i: (0, 0)),
            pl.BlockSpec((1, 1), lambda i: (0, 0)),
        ],
        out_specs=pl.BlockSpec((R, 1), lambda i: (i, 0)),
        out_shape=jax.ShapeDtypeStruct((B, 1), jnp.float32),
    )(ue, me, w1, c1, w2, c2, w3, c3)
    return out
